# baseline (speedup 1.0000x reference)
import functools

import jax
import jax.numpy as jnp
from jax import lax
from jax.experimental import pallas as pl
from jax.experimental.pallas import tpu as pltpu

INPUT_DIM = 1024
LATENT_DIM = 64
NUM_EMBEDDINGS = 1024
BATCH = 16384

TILE = 512


def _vq_body(x_ref, we_ref, be_ref, emb_ref, wd_ref, bd_ref, out_ref):
    x = x_ref[...]
    z = jnp.dot(x, we_ref[...], preferred_element_type=jnp.float32) + be_ref[...]
    sim = jnp.dot(z, emb_ref[...], preferred_element_type=jnp.float32)
    e2 = jnp.sum(emb_ref[...] ** 2, axis=0, keepdims=True)
    d = jnp.sum(z * z, axis=1, keepdims=True) + e2 - 2.0 * sim
    idx = jnp.argmin(d, axis=1)
    enc = (lax.broadcasted_iota(jnp.int32, (TILE, NUM_EMBEDDINGS), 1)
           == idx[:, None]).astype(jnp.float32)
    q = lax.dot_general(enc, emb_ref[...], (((1,), (1,)), ((), ())),
                        preferred_element_type=jnp.float32)
    out_ref[...] = (jnp.dot(q, wd_ref[...], preferred_element_type=jnp.float32)
                    + bd_ref[...])


@jax.jit
def kernel(x, W_enc, b_enc, W_emb, W_dec, b_dec):
    nb = BATCH // TILE
    full = lambda shape: pl.BlockSpec(shape, lambda i: (0,) * len(shape))
    out = pl.pallas_call(
        _vq_body,
        grid=(nb,),
        in_specs=[
            pl.BlockSpec((TILE, INPUT_DIM), lambda i: (i, 0)),
            full((INPUT_DIM, LATENT_DIM)),
            full((1, LATENT_DIM)),
            full((LATENT_DIM, NUM_EMBEDDINGS)),
            full((LATENT_DIM, INPUT_DIM)),
            full((1, INPUT_DIM)),
        ],
        out_specs=pl.BlockSpec((TILE, INPUT_DIM), lambda i: (i, 0)),
        out_shape=jax.ShapeDtypeStruct((BATCH, INPUT_DIM), jnp.float32),
    )(x, W_enc, b_enc.reshape(1, -1), W_emb, W_dec, b_dec.reshape(1, -1))
    return out
